# baseline (speedup 1.0000x reference)
import functools

import jax
import jax.numpy as jnp
from jax import lax
from jax.experimental import pallas as pl
from jax.experimental.pallas import tpu as pltpu
from jax.experimental.pallas import tpu_sc as plsc

NUM_ACTIONS = 64
LANES = 16

_GATHER_DNUMS = lax.GatherDimensionNumbers(
    offset_dims=(), collapsed_slice_dims=(0,), start_index_map=(0,))


def _lane_gather(x, idx):
    return lax.gather(x, idx[:, None], _GATHER_DNUMS, (1,),
                      mode=lax.GatherScatterMode.PROMISE_IN_BOUNDS)


def kernel(logits, s_idx):
    num_states = logits.shape[0]
    batch = s_idx.shape[0]
    info = plsc.get_sparse_core_info()
    nc, ns = info.num_cores, info.num_subcores
    nw = nc * ns
    b_per_w = batch // nw

    idx2d = s_idx.reshape(nw, b_per_w)
    mesh = plsc.VectorSubcoreMesh(core_axis_name="c", subcore_axis_name="s")

    @functools.partial(
        pl.kernel,
        mesh=mesh,
        out_type=jax.ShapeDtypeStruct((batch, NUM_ACTIONS), jnp.float32),
        scratch_types=[
            pltpu.VMEM((b_per_w,), jnp.int32),
            pltpu.VMEM((b_per_w, NUM_ACTIONS), jnp.float32),
            pltpu.SemaphoreType.DMA,
        ],
        compiler_params=pltpu.CompilerParams(
            needs_layout_passes=False, use_tc_tiling_on_sc=False),
    )
    def sc_gather_softmax(table_hbm, idx_hbm, out_hbm, idx_v, rows_v, sem):
        wid = lax.axis_index("s") * nc + lax.axis_index("c")
        pltpu.sync_copy(idx_hbm.at[wid], idx_v)
        pltpu.async_copy(table_hbm.at[idx_v], rows_v, sem).wait()

        last = jnp.full((LANES,), LANES - 1, dtype=jnp.int32)

        def row_body(r, carry):
            x0 = rows_v[r, pl.ds(0, LANES)]
            x1 = rows_v[r, pl.ds(LANES, LANES)]
            x2 = rows_v[r, pl.ds(2 * LANES, LANES)]
            x3 = rows_v[r, pl.ds(3 * LANES, LANES)]
            m16 = jnp.maximum(jnp.maximum(x0, x1), jnp.maximum(x2, x3))
            m = _lane_gather(plsc.cummax(m16), last)
            e0 = jnp.exp(x0 - m)
            e1 = jnp.exp(x1 - m)
            e2 = jnp.exp(x2 - m)
            e3 = jnp.exp(x3 - m)
            s16 = (e0 + e1) + (e2 + e3)
            s = _lane_gather(plsc.cumsum(s16), last)
            inv = 1.0 / s
            rows_v[r, pl.ds(0, LANES)] = e0 * inv
            rows_v[r, pl.ds(LANES, LANES)] = e1 * inv
            rows_v[r, pl.ds(2 * LANES, LANES)] = e2 * inv
            rows_v[r, pl.ds(3 * LANES, LANES)] = e3 * inv
            return carry

        lax.fori_loop(0, b_per_w, row_body, 0)
        pltpu.sync_copy(rows_v, out_hbm.at[pl.ds(wid * b_per_w, b_per_w)])

    return sc_gather_softmax(logits, idx2d)


# speedup vs baseline: 1.0196x; 1.0196x over previous
import functools

import jax
import jax.numpy as jnp
from jax import lax
from jax.experimental import pallas as pl
from jax.experimental.pallas import tpu as pltpu
from jax.experimental.pallas import tpu_sc as plsc

NUM_ACTIONS = 64
LANES = 16

_GATHER_DNUMS = lax.GatherDimensionNumbers(
    offset_dims=(), collapsed_slice_dims=(0,), start_index_map=(0,))


def _lane_gather(x, idx):
    return lax.gather(x, idx[:, None], _GATHER_DNUMS, (1,),
                      mode=lax.GatherScatterMode.PROMISE_IN_BOUNDS)


def kernel(logits, s_idx):
    num_states = logits.shape[0]
    batch = s_idx.shape[0]
    info = plsc.get_sparse_core_info()
    nc, ns = info.num_cores, info.num_subcores
    nw = nc * ns
    b_per_w = batch // nw

    idx2d = s_idx.reshape(nw, b_per_w)
    mesh = plsc.VectorSubcoreMesh(core_axis_name="c", subcore_axis_name="s")

    @functools.partial(
        pl.kernel,
        mesh=mesh,
        out_type=jax.ShapeDtypeStruct((batch, NUM_ACTIONS), jnp.float32),
        scratch_types=[
            pltpu.VMEM((b_per_w,), jnp.int32),
            pltpu.VMEM((b_per_w, NUM_ACTIONS), jnp.float32),
            pltpu.SemaphoreType.DMA,
        ],
        compiler_params=pltpu.CompilerParams(
            needs_layout_passes=False, use_tc_tiling_on_sc=False),
    )
    def sc_gather_softmax(table_hbm, idx_hbm, out_hbm, idx_v, rows_v, sem):
        wid = lax.axis_index("s") * nc + lax.axis_index("c")
        pltpu.sync_copy(idx_hbm.at[wid], idx_v)
        pltpu.async_copy(table_hbm.at[idx_v], rows_v, sem).wait()

        last = jnp.full((LANES,), LANES - 1, dtype=jnp.int32)

        @plsc.parallel_loop(0, b_per_w, step=1, unroll=8)
        def row_body(r):
            x0 = rows_v[r, pl.ds(0, LANES)]
            x1 = rows_v[r, pl.ds(LANES, LANES)]
            x2 = rows_v[r, pl.ds(2 * LANES, LANES)]
            x3 = rows_v[r, pl.ds(3 * LANES, LANES)]
            m16 = jnp.maximum(jnp.maximum(x0, x1), jnp.maximum(x2, x3))
            m = _lane_gather(plsc.cummax(m16), last)
            e0 = jnp.exp(x0 - m)
            e1 = jnp.exp(x1 - m)
            e2 = jnp.exp(x2 - m)
            e3 = jnp.exp(x3 - m)
            s16 = (e0 + e1) + (e2 + e3)
            s = _lane_gather(plsc.cumsum(s16), last)
            inv = 1.0 / s
            rows_v[r, pl.ds(0, LANES)] = e0 * inv
            rows_v[r, pl.ds(LANES, LANES)] = e1 * inv
            rows_v[r, pl.ds(2 * LANES, LANES)] = e2 * inv
            rows_v[r, pl.ds(3 * LANES, LANES)] = e3 * inv

        pltpu.sync_copy(rows_v, out_hbm.at[pl.ds(wid * b_per_w, b_per_w)])

    return sc_gather_softmax(logits, idx2d)
